# baseline (speedup 1.0000x reference)
import functools

import jax
import jax.numpy as jnp
from jax import lax
from jax.experimental import pallas as pl
from jax.experimental.pallas import tpu as pltpu
from jax.experimental.pallas import tpu_sc as plsc

N = 10000
E = 160000
D_IN = 1433
D_HID = 32

NC = 2
NS = 16
NW = NC * NS

CHUNK = 128
CPW = 40
E_PAD = NW * CPW * CHUNK
ACC_ROWS = 10112
ZROWS = ACC_ROWS // NS
OSTRIPE = 624
NBUF = 4


def _make_sc_agg(with_counts):
    mesh = plsc.VectorSubcoreMesh(core_axis_name="c", subcore_axis_name="s")

    out_type = [jax.ShapeDtypeStruct((2 * N, D_HID), jnp.float32)]
    scratch = [
        pltpu.VMEM((CPW, CHUNK), jnp.int32),
        pltpu.VMEM((CPW, CHUNK), jnp.int32),
        [pltpu.VMEM((CHUNK, D_HID), jnp.float32) for _ in range(NBUF)],
        pltpu.VMEM_SHARED((ACC_ROWS, D_HID), jnp.float32),
        [pltpu.SemaphoreType.DMA for _ in range(NBUF)],
        [pltpu.SemaphoreType.DMA for _ in range(NBUF)],
    ]
    if with_counts:
        out_type.append(jax.ShapeDtypeStruct((2 * N, 16), jnp.float32))
        scratch += [
            pltpu.VMEM((CHUNK, 16), jnp.float32),
            pltpu.VMEM((CHUNK, 16), jnp.float32),
            pltpu.VMEM_SHARED((ACC_ROWS, 16), jnp.float32),
            [pltpu.SemaphoreType.DMA for _ in range(NBUF)],
        ]

    @functools.partial(
        pl.kernel,
        out_type=out_type,
        mesh=mesh,
        scratch_types=scratch,
        compiler_params=pltpu.CompilerParams(use_tc_tiling_on_sc=False),
    )
    def agg(table_hbm, srcs_hbm, dsts_hbm, *rest):
        if with_counts:
            (f_out, c_out, src_v, dst_v, rows, acc_sh, gsem, ssem,
             ones_v, z16_v, acc_cnt, csem) = rest
        else:
            f_out, src_v, dst_v, rows, acc_sh, gsem, ssem = rest
        cid = lax.axis_index("c")
        sid = lax.axis_index("s")
        wid = sid * NC + cid
        base = sid * ZROWS

        zv = jnp.zeros((16,), jnp.float32)

        def zrow(r, c):
            rows[0][r, pl.ds(0, 16)] = zv
            rows[0][r, pl.ds(16, 16)] = zv
            if with_counts:
                ones_v[r, pl.ds(0, 16)] = jnp.ones((16,), jnp.float32)
                z16_v[r, pl.ds(0, 16)] = zv
            return c

        lax.fori_loop(0, CHUNK, zrow, 0)
        for k in range(4):
            pltpu.sync_copy(rows[0], acc_sh.at[pl.ds(base + k * CHUNK, CHUNK)])
        pltpu.sync_copy(rows[0].at[pl.ds(0, ZROWS - 4 * CHUNK)],
                        acc_sh.at[pl.ds(base + 4 * CHUNK, ZROWS - 4 * CHUNK)])
        if with_counts:
            for k in range(4):
                pltpu.sync_copy(z16_v, acc_cnt.at[pl.ds(base + k * CHUNK, CHUNK)])
            pltpu.sync_copy(z16_v.at[pl.ds(0, ZROWS - 4 * CHUNK)],
                            acc_cnt.at[pl.ds(base + 4 * CHUNK, ZROWS - 4 * CHUNK)])

        pltpu.sync_copy(srcs_hbm.at[wid], src_v)
        pltpu.sync_copy(dsts_hbm.at[wid], dst_v)
        plsc.subcore_barrier()

        for b in range(NBUF):
            pltpu.async_copy(table_hbm.at[src_v.at[b]], rows[b], gsem[b])

        G = CPW // NBUF

        def body(g, carry):
            j0 = g * NBUF
            for b in range(NBUF):
                pltpu.make_async_copy(
                    table_hbm.at[src_v.at[0]], rows[b], gsem[b]).wait()
                pltpu.async_copy(
                    rows[b], acc_sh.at[dst_v.at[j0 + b]], ssem[b], add=True)
                if with_counts:
                    pltpu.async_copy(
                        ones_v, acc_cnt.at[dst_v.at[j0 + b]], csem[b], add=True)

            @pl.when(g < G - 1)
            def _refill():
                for b in range(NBUF):
                    pltpu.make_async_copy(
                        rows[b], acc_sh.at[dst_v.at[0]], ssem[b]).wait()
                    if with_counts:
                        pltpu.make_async_copy(
                            ones_v, acc_cnt.at[dst_v.at[0]], csem[b]).wait()
                    pltpu.async_copy(
                        table_hbm.at[src_v.at[j0 + NBUF + b]], rows[b], gsem[b])
            return carry

        lax.fori_loop(0, G, body, 0)
        for b in range(NBUF):
            pltpu.make_async_copy(rows[b], acc_sh.at[dst_v.at[0]], ssem[b]).wait()
            if with_counts:
                pltpu.make_async_copy(
                    ones_v, acc_cnt.at[dst_v.at[0]], csem[b]).wait()
        plsc.subcore_barrier()

        last = (NS - 1) * OSTRIPE

        @pl.when(sid < NS - 1)
        def _copy_main():
            pltpu.sync_copy(acc_sh.at[pl.ds(sid * OSTRIPE, OSTRIPE)],
                            f_out.at[pl.ds(cid * N + sid * OSTRIPE, OSTRIPE)])
            if with_counts:
                pltpu.sync_copy(acc_cnt.at[pl.ds(sid * OSTRIPE, OSTRIPE)],
                                c_out.at[pl.ds(cid * N + sid * OSTRIPE, OSTRIPE)])

        @pl.when(sid == NS - 1)
        def _copy_tail():
            pltpu.sync_copy(acc_sh.at[pl.ds(last, N - last)],
                            f_out.at[pl.ds(cid * N + last, N - last)])
            if with_counts:
                pltpu.sync_copy(acc_cnt.at[pl.ds(last, N - last)],
                                c_out.at[pl.ds(cid * N + last, N - last)])

    return agg


_sc_agg_cnt = _make_sc_agg(True)
_sc_agg = _make_sc_agg(False)

_BN = 1000


def _tc_a(x, wlt, wrt):
    def body(x_ref, wl_ref, wr_ref, p1_ref, r1_ref):
        xb = x_ref[...]
        p1_ref[...] = jnp.dot(xb, wl_ref[...], preferred_element_type=jnp.float32)
        r1_ref[...] = jnp.dot(xb, wr_ref[...], preferred_element_type=jnp.float32)

    return pl.pallas_call(
        body,
        grid=(N // _BN,),
        in_specs=[pl.BlockSpec((_BN, D_IN), lambda i: (i, 0)),
                  pl.BlockSpec((D_IN, D_HID), lambda i: (0, 0)),
                  pl.BlockSpec((D_IN, D_HID), lambda i: (0, 0))],
        out_specs=[pl.BlockSpec((_BN, D_HID), lambda i: (i, 0)),
                   pl.BlockSpec((_BN, D_HID), lambda i: (i, 0))],
        out_shape=[jax.ShapeDtypeStruct((N, D_HID), jnp.float32),
                   jax.ShapeDtypeStruct((N, D_HID), jnp.float32)],
    )(x, wlt, wrt)


def _tc_c(parts1, cnts, r1, b1, w2lt, w2rt, b2):
    def body(p0_ref, p1_ref, c0_ref, c1_ref, r1_ref, b1_ref, wl_ref, wr_ref,
             b2_ref, p2_ref, r2_ref, inv_ref):
        s = p0_ref[...] + p1_ref[...]
        cnt = c0_ref[:, 0:1] + c1_ref[:, 0:1]
        inv = 1.0 / jnp.maximum(cnt, 1.0)
        h1 = s * inv + b1_ref[...] + r1_ref[...]
        p2_ref[...] = jnp.dot(h1, wl_ref[...], preferred_element_type=jnp.float32)
        r2_ref[...] = jnp.dot(h1, wr_ref[...],
                              preferred_element_type=jnp.float32) + b2_ref[...]
        inv_ref[...] = inv

    nb = N // _BN
    return pl.pallas_call(
        body,
        grid=(nb,),
        in_specs=[pl.BlockSpec((_BN, D_HID), lambda i: (i, 0)),
                  pl.BlockSpec((_BN, D_HID), lambda i: (i + nb, 0)),
                  pl.BlockSpec((_BN, 16), lambda i: (i, 0)),
                  pl.BlockSpec((_BN, 16), lambda i: (i + nb, 0)),
                  pl.BlockSpec((_BN, D_HID), lambda i: (i, 0)),
                  pl.BlockSpec((1, D_HID), lambda i: (0, 0)),
                  pl.BlockSpec((D_HID, D_HID), lambda i: (0, 0)),
                  pl.BlockSpec((D_HID, D_HID), lambda i: (0, 0)),
                  pl.BlockSpec((1, D_HID), lambda i: (0, 0))],
        out_specs=[pl.BlockSpec((_BN, D_HID), lambda i: (i, 0)),
                   pl.BlockSpec((_BN, D_HID), lambda i: (i, 0)),
                   pl.BlockSpec((_BN, 1), lambda i: (i, 0))],
        out_shape=[jax.ShapeDtypeStruct((N, D_HID), jnp.float32),
                   jax.ShapeDtypeStruct((N, D_HID), jnp.float32),
                   jax.ShapeDtypeStruct((N, 1), jnp.float32)],
    )(parts1, parts1, cnts, cnts, r1, b1, w2lt, w2rt, b2)


def _tc_e(parts2, r2b, inv):
    nb = N // _BN

    def body(p0_ref, p1_ref, r2_ref, inv_ref, out_ref):
        h2 = (p0_ref[...] + p1_ref[...]) * inv_ref[...] + r2_ref[...]
        h2 = jnp.maximum(h2, 0.0)
        m = jnp.max(h2, axis=1, keepdims=True)
        lse = jnp.log(jnp.sum(jnp.exp(h2 - m), axis=1, keepdims=True)) + m
        out_ref[...] = h2 - lse

    return pl.pallas_call(
        body,
        grid=(nb,),
        in_specs=[pl.BlockSpec((_BN, D_HID), lambda i: (i, 0)),
                  pl.BlockSpec((_BN, D_HID), lambda i: (i + nb, 0)),
                  pl.BlockSpec((_BN, D_HID), lambda i: (i, 0)),
                  pl.BlockSpec((_BN, 1), lambda i: (i, 0))],
        out_specs=pl.BlockSpec((_BN, D_HID), lambda i: (i, 0)),
        out_shape=jax.ShapeDtypeStruct((N, D_HID), jnp.float32),
    )(parts2, parts2, r2b, inv)


def kernel(x, edge_index, W1l, b1l, W1r, W2l, b2l, W2r):
    src = edge_index[0]
    dst = edge_index[1]
    pad = E_PAD - E
    pad_src = jnp.arange(pad, dtype=jnp.int32) % N
    srcs = jnp.concatenate([src, pad_src]).reshape(NW, CPW, CHUNK)
    pad_dst = N + (jnp.arange(pad, dtype=jnp.int32) % (ACC_ROWS - N))
    dsts = jnp.concatenate([dst, pad_dst]).reshape(NW, CPW, CHUNK)

    a1, r1 = _tc_a(x, W1l.T, W1r.T)
    parts1, cnts = _sc_agg_cnt(a1, srcs, dsts)
    p2, r2b, inv = _tc_c(parts1, cnts, r1, b1l.reshape(1, D_HID),
                         W2l.T, W2r.T, b2l.reshape(1, D_HID))
    parts2, = _sc_agg(p2, srcs, dsts)
    return _tc_e(parts2, r2b, inv)


# speedup vs baseline: 1.0154x; 1.0154x over previous
import functools

import jax
import jax.numpy as jnp
from jax import lax
from jax.experimental import pallas as pl
from jax.experimental.pallas import tpu as pltpu
from jax.experimental.pallas import tpu_sc as plsc

N = 10000
E = 160000
D_IN = 1433
D_HID = 32

NC = 2
NS = 16
NW = NC * NS

CHUNK = 128
CPW = 40
E_PAD = NW * CPW * CHUNK
ACC_ROWS = 10112
ZROWS = ACC_ROWS // NS
OSTRIPE = 624
NBUF = 8


def _make_sc_agg(with_counts):
    mesh = plsc.VectorSubcoreMesh(core_axis_name="c", subcore_axis_name="s")

    out_type = [jax.ShapeDtypeStruct((2 * N, D_HID), jnp.float32)]
    scratch = [
        pltpu.VMEM((CPW, CHUNK), jnp.int32),
        pltpu.VMEM((CPW, CHUNK), jnp.int32),
        [pltpu.VMEM((CHUNK, D_HID), jnp.float32) for _ in range(NBUF)],
        pltpu.VMEM_SHARED((ACC_ROWS, D_HID), jnp.float32),
        [pltpu.SemaphoreType.DMA for _ in range(NBUF)],
        [pltpu.SemaphoreType.DMA for _ in range(NBUF)],
    ]
    if with_counts:
        out_type.append(jax.ShapeDtypeStruct((2 * N, 16), jnp.float32))
        scratch += [
            pltpu.VMEM((CHUNK, 16), jnp.float32),
            pltpu.VMEM((CHUNK, 16), jnp.float32),
            pltpu.VMEM_SHARED((ACC_ROWS, 16), jnp.float32),
            [pltpu.SemaphoreType.DMA for _ in range(NBUF)],
        ]

    @functools.partial(
        pl.kernel,
        out_type=out_type,
        mesh=mesh,
        scratch_types=scratch,
        compiler_params=pltpu.CompilerParams(use_tc_tiling_on_sc=False),
    )
    def agg(table_hbm, srcs_hbm, dsts_hbm, *rest):
        if with_counts:
            (f_out, c_out, src_v, dst_v, rows, acc_sh, gsem, ssem,
             ones_v, z16_v, acc_cnt, csem) = rest
        else:
            f_out, src_v, dst_v, rows, acc_sh, gsem, ssem = rest
        cid = lax.axis_index("c")
        sid = lax.axis_index("s")
        wid = sid * NC + cid
        base = sid * ZROWS

        zv = jnp.zeros((16,), jnp.float32)

        def zrow(r, c):
            rows[0][r, pl.ds(0, 16)] = zv
            rows[0][r, pl.ds(16, 16)] = zv
            if with_counts:
                ones_v[r, pl.ds(0, 16)] = jnp.ones((16,), jnp.float32)
                z16_v[r, pl.ds(0, 16)] = zv
            return c

        lax.fori_loop(0, CHUNK, zrow, 0)
        for k in range(4):
            pltpu.sync_copy(rows[0], acc_sh.at[pl.ds(base + k * CHUNK, CHUNK)])
        pltpu.sync_copy(rows[0].at[pl.ds(0, ZROWS - 4 * CHUNK)],
                        acc_sh.at[pl.ds(base + 4 * CHUNK, ZROWS - 4 * CHUNK)])
        if with_counts:
            for k in range(4):
                pltpu.sync_copy(z16_v, acc_cnt.at[pl.ds(base + k * CHUNK, CHUNK)])
            pltpu.sync_copy(z16_v.at[pl.ds(0, ZROWS - 4 * CHUNK)],
                            acc_cnt.at[pl.ds(base + 4 * CHUNK, ZROWS - 4 * CHUNK)])

        pltpu.sync_copy(srcs_hbm.at[wid], src_v)
        pltpu.sync_copy(dsts_hbm.at[wid], dst_v)
        plsc.subcore_barrier()

        for b in range(NBUF):
            pltpu.async_copy(table_hbm.at[src_v.at[b]], rows[b], gsem[b])

        G = CPW // NBUF

        def body(g, carry):
            j0 = g * NBUF
            for b in range(NBUF):
                pltpu.make_async_copy(
                    table_hbm.at[src_v.at[0]], rows[b], gsem[b]).wait()
                pltpu.async_copy(
                    rows[b], acc_sh.at[dst_v.at[j0 + b]], ssem[b], add=True)
                if with_counts:
                    pltpu.async_copy(
                        ones_v, acc_cnt.at[dst_v.at[j0 + b]], csem[b], add=True)

            @pl.when(g < G - 1)
            def _refill():
                for b in range(NBUF):
                    pltpu.make_async_copy(
                        rows[b], acc_sh.at[dst_v.at[0]], ssem[b]).wait()
                    if with_counts:
                        pltpu.make_async_copy(
                            ones_v, acc_cnt.at[dst_v.at[0]], csem[b]).wait()
                    pltpu.async_copy(
                        table_hbm.at[src_v.at[j0 + NBUF + b]], rows[b], gsem[b])
            return carry

        lax.fori_loop(0, G, body, 0)
        for b in range(NBUF):
            pltpu.make_async_copy(rows[b], acc_sh.at[dst_v.at[0]], ssem[b]).wait()
            if with_counts:
                pltpu.make_async_copy(
                    ones_v, acc_cnt.at[dst_v.at[0]], csem[b]).wait()
        plsc.subcore_barrier()

        last = (NS - 1) * OSTRIPE

        @pl.when(sid < NS - 1)
        def _copy_main():
            pltpu.sync_copy(acc_sh.at[pl.ds(sid * OSTRIPE, OSTRIPE)],
                            f_out.at[pl.ds(cid * N + sid * OSTRIPE, OSTRIPE)])
            if with_counts:
                pltpu.sync_copy(acc_cnt.at[pl.ds(sid * OSTRIPE, OSTRIPE)],
                                c_out.at[pl.ds(cid * N + sid * OSTRIPE, OSTRIPE)])

        @pl.when(sid == NS - 1)
        def _copy_tail():
            pltpu.sync_copy(acc_sh.at[pl.ds(last, N - last)],
                            f_out.at[pl.ds(cid * N + last, N - last)])
            if with_counts:
                pltpu.sync_copy(acc_cnt.at[pl.ds(last, N - last)],
                                c_out.at[pl.ds(cid * N + last, N - last)])

    return agg


_sc_agg_cnt = _make_sc_agg(True)
_sc_agg = _make_sc_agg(False)

_BN = 1000


def _tc_a(x, wlt, wrt):
    def body(x_ref, wl_ref, wr_ref, p1_ref, r1_ref):
        xb = x_ref[...]
        p1_ref[...] = jnp.dot(xb, wl_ref[...], preferred_element_type=jnp.float32)
        r1_ref[...] = jnp.dot(xb, wr_ref[...], preferred_element_type=jnp.float32)

    return pl.pallas_call(
        body,
        grid=(N // _BN,),
        in_specs=[pl.BlockSpec((_BN, D_IN), lambda i: (i, 0)),
                  pl.BlockSpec((D_IN, D_HID), lambda i: (0, 0)),
                  pl.BlockSpec((D_IN, D_HID), lambda i: (0, 0))],
        out_specs=[pl.BlockSpec((_BN, D_HID), lambda i: (i, 0)),
                   pl.BlockSpec((_BN, D_HID), lambda i: (i, 0))],
        out_shape=[jax.ShapeDtypeStruct((N, D_HID), jnp.float32),
                   jax.ShapeDtypeStruct((N, D_HID), jnp.float32)],
    )(x, wlt, wrt)


def _tc_c(parts1, cnts, r1, b1, w2lt, w2rt, b2):
    def body(p0_ref, p1_ref, c0_ref, c1_ref, r1_ref, b1_ref, wl_ref, wr_ref,
             b2_ref, p2_ref, r2_ref, inv_ref):
        s = p0_ref[...] + p1_ref[...]
        cnt = c0_ref[:, 0:1] + c1_ref[:, 0:1]
        inv = 1.0 / jnp.maximum(cnt, 1.0)
        h1 = s * inv + b1_ref[...] + r1_ref[...]
        p2_ref[...] = jnp.dot(h1, wl_ref[...], preferred_element_type=jnp.float32)
        r2_ref[...] = jnp.dot(h1, wr_ref[...],
                              preferred_element_type=jnp.float32) + b2_ref[...]
        inv_ref[...] = inv

    nb = N // _BN
    return pl.pallas_call(
        body,
        grid=(nb,),
        in_specs=[pl.BlockSpec((_BN, D_HID), lambda i: (i, 0)),
                  pl.BlockSpec((_BN, D_HID), lambda i: (i + nb, 0)),
                  pl.BlockSpec((_BN, 16), lambda i: (i, 0)),
                  pl.BlockSpec((_BN, 16), lambda i: (i + nb, 0)),
                  pl.BlockSpec((_BN, D_HID), lambda i: (i, 0)),
                  pl.BlockSpec((1, D_HID), lambda i: (0, 0)),
                  pl.BlockSpec((D_HID, D_HID), lambda i: (0, 0)),
                  pl.BlockSpec((D_HID, D_HID), lambda i: (0, 0)),
                  pl.BlockSpec((1, D_HID), lambda i: (0, 0))],
        out_specs=[pl.BlockSpec((_BN, D_HID), lambda i: (i, 0)),
                   pl.BlockSpec((_BN, D_HID), lambda i: (i, 0)),
                   pl.BlockSpec((_BN, 1), lambda i: (i, 0))],
        out_shape=[jax.ShapeDtypeStruct((N, D_HID), jnp.float32),
                   jax.ShapeDtypeStruct((N, D_HID), jnp.float32),
                   jax.ShapeDtypeStruct((N, 1), jnp.float32)],
    )(parts1, parts1, cnts, cnts, r1, b1, w2lt, w2rt, b2)


def _tc_e(parts2, r2b, inv):
    nb = N // _BN

    def body(p0_ref, p1_ref, r2_ref, inv_ref, out_ref):
        h2 = (p0_ref[...] + p1_ref[...]) * inv_ref[...] + r2_ref[...]
        h2 = jnp.maximum(h2, 0.0)
        m = jnp.max(h2, axis=1, keepdims=True)
        lse = jnp.log(jnp.sum(jnp.exp(h2 - m), axis=1, keepdims=True)) + m
        out_ref[...] = h2 - lse

    return pl.pallas_call(
        body,
        grid=(nb,),
        in_specs=[pl.BlockSpec((_BN, D_HID), lambda i: (i, 0)),
                  pl.BlockSpec((_BN, D_HID), lambda i: (i + nb, 0)),
                  pl.BlockSpec((_BN, D_HID), lambda i: (i, 0)),
                  pl.BlockSpec((_BN, 1), lambda i: (i, 0))],
        out_specs=pl.BlockSpec((_BN, D_HID), lambda i: (i, 0)),
        out_shape=jax.ShapeDtypeStruct((N, D_HID), jnp.float32),
    )(parts2, parts2, r2b, inv)


def kernel(x, edge_index, W1l, b1l, W1r, W2l, b2l, W2r):
    src = edge_index[0]
    dst = edge_index[1]
    pad = E_PAD - E
    pad_src = jnp.arange(pad, dtype=jnp.int32) % N
    srcs = jnp.concatenate([src, pad_src]).reshape(NW, CPW, CHUNK)
    pad_dst = N + (jnp.arange(pad, dtype=jnp.int32) % (ACC_ROWS - N))
    dsts = jnp.concatenate([dst, pad_dst]).reshape(NW, CPW, CHUNK)

    a1, r1 = _tc_a(x, W1l.T, W1r.T)
    parts1, cnts = _sc_agg_cnt(a1, srcs, dsts)
    p2, r2b, inv = _tc_c(parts1, cnts, r1, b1l.reshape(1, D_HID),
                         W2l.T, W2r.T, b2l.reshape(1, D_HID))
    parts2, = _sc_agg(p2, srcs, dsts)
    return _tc_e(parts2, r2b, inv)


# speedup vs baseline: 1.0471x; 1.0312x over previous
import functools

import jax
import jax.numpy as jnp
from jax import lax
from jax.experimental import pallas as pl
from jax.experimental.pallas import tpu as pltpu
from jax.experimental.pallas import tpu_sc as plsc

N = 10000
E = 160000
D_IN = 1433
D_HID = 32

NC = 2
NS = 16
NW = NC * NS

CHUNK = 128
NCHUNKS = E // CHUNK
CPW = 40
CPW_LAST = NCHUNKS - (NW - 1) * CPW
NBUF = 10
ACC_ROWS = 10112
ZROWS = ACC_ROWS // NS
OSTRIPE = 624


def _make_sc_agg(with_counts):
    mesh = plsc.VectorSubcoreMesh(core_axis_name="c", subcore_axis_name="s")

    out_type = [jax.ShapeDtypeStruct((2 * N, D_HID), jnp.float32)]
    scratch = [
        pltpu.VMEM((CPW, CHUNK), jnp.int32),
        pltpu.VMEM((CPW, CHUNK), jnp.int32),
        [pltpu.VMEM((CHUNK, D_HID), jnp.float32) for _ in range(NBUF)],
        pltpu.VMEM_SHARED((ACC_ROWS, D_HID), jnp.float32),
        [pltpu.SemaphoreType.DMA for _ in range(NBUF)],
        [pltpu.SemaphoreType.DMA for _ in range(NBUF)],
    ]
    if with_counts:
        out_type.append(jax.ShapeDtypeStruct((2 * N, 16), jnp.float32))
        scratch += [
            pltpu.VMEM((CHUNK, 16), jnp.float32),
            pltpu.VMEM((CHUNK, 16), jnp.float32),
            pltpu.VMEM_SHARED((ACC_ROWS, 16), jnp.float32),
            [pltpu.SemaphoreType.DMA for _ in range(NBUF)],
        ]

    @functools.partial(
        pl.kernel,
        out_type=out_type,
        mesh=mesh,
        scratch_types=scratch,
        compiler_params=pltpu.CompilerParams(use_tc_tiling_on_sc=False),
    )
    def agg(table_hbm, ei_hbm, *rest):
        if with_counts:
            (f_out, c_out, src_v, dst_v, rows, acc_sh, gsem, ssem,
             ones_v, z16_v, acc_cnt, csem) = rest
        else:
            f_out, src_v, dst_v, rows, acc_sh, gsem, ssem = rest
        cid = lax.axis_index("c")
        sid = lax.axis_index("s")
        wid = sid * NC + cid
        base = sid * ZROWS

        zv = jnp.zeros((16,), jnp.float32)

        def zrow(r, c):
            rows[0][r, pl.ds(0, 16)] = zv
            rows[0][r, pl.ds(16, 16)] = zv
            if with_counts:
                ones_v[r, pl.ds(0, 16)] = jnp.ones((16,), jnp.float32)
                z16_v[r, pl.ds(0, 16)] = zv
            return c

        lax.fori_loop(0, CHUNK, zrow, 0)
        for k in range(4):
            pltpu.sync_copy(rows[0], acc_sh.at[pl.ds(base + k * CHUNK, CHUNK)])
        pltpu.sync_copy(rows[0].at[pl.ds(0, ZROWS - 4 * CHUNK)],
                        acc_sh.at[pl.ds(base + 4 * CHUNK, ZROWS - 4 * CHUNK)])
        if with_counts:
            for k in range(4):
                pltpu.sync_copy(z16_v, acc_cnt.at[pl.ds(base + k * CHUNK, CHUNK)])
            pltpu.sync_copy(z16_v.at[pl.ds(0, ZROWS - 4 * CHUNK)],
                            acc_cnt.at[pl.ds(base + 4 * CHUNK, ZROWS - 4 * CHUNK)])

        @pl.when(wid < NW - 1)
        def _stage_full():
            pltpu.sync_copy(ei_hbm.at[0, pl.ds(wid * CPW, CPW)], src_v)
            pltpu.sync_copy(ei_hbm.at[1, pl.ds(wid * CPW, CPW)], dst_v)

        @pl.when(wid == NW - 1)
        def _stage_tail():
            pltpu.sync_copy(ei_hbm.at[0, pl.ds((NW - 1) * CPW, CPW_LAST)],
                            src_v.at[pl.ds(0, CPW_LAST)])
            pltpu.sync_copy(ei_hbm.at[1, pl.ds((NW - 1) * CPW, CPW_LAST)],
                            dst_v.at[pl.ds(0, CPW_LAST)])

        plsc.subcore_barrier()

        for b in range(NBUF):
            pltpu.async_copy(table_hbm.at[src_v.at[b]], rows[b], gsem[b])

        n_groups = jnp.where(wid == NW - 1, 1, CPW // NBUF)

        def body(g, carry):
            j0 = g * NBUF
            for b in range(NBUF):
                pltpu.make_async_copy(
                    table_hbm.at[src_v.at[0]], rows[b], gsem[b]).wait()
                pltpu.async_copy(
                    rows[b], acc_sh.at[dst_v.at[j0 + b]], ssem[b], add=True)
                if with_counts:
                    pltpu.async_copy(
                        ones_v, acc_cnt.at[dst_v.at[j0 + b]], csem[b], add=True)

            @pl.when(g < n_groups - 1)
            def _refill():
                for b in range(NBUF):
                    pltpu.make_async_copy(
                        rows[b], acc_sh.at[dst_v.at[0]], ssem[b]).wait()
                    if with_counts:
                        pltpu.make_async_copy(
                            ones_v, acc_cnt.at[dst_v.at[0]], csem[b]).wait()
                    pltpu.async_copy(
                        table_hbm.at[src_v.at[j0 + NBUF + b]], rows[b], gsem[b])
            return carry

        lax.fori_loop(0, n_groups, body, 0)
        for b in range(NBUF):
            pltpu.make_async_copy(rows[b], acc_sh.at[dst_v.at[0]], ssem[b]).wait()
            if with_counts:
                pltpu.make_async_copy(
                    ones_v, acc_cnt.at[dst_v.at[0]], csem[b]).wait()
        plsc.subcore_barrier()

        last = (NS - 1) * OSTRIPE

        @pl.when(sid < NS - 1)
        def _copy_main():
            pltpu.sync_copy(acc_sh.at[pl.ds(sid * OSTRIPE, OSTRIPE)],
                            f_out.at[pl.ds(cid * N + sid * OSTRIPE, OSTRIPE)])
            if with_counts:
                pltpu.sync_copy(acc_cnt.at[pl.ds(sid * OSTRIPE, OSTRIPE)],
                                c_out.at[pl.ds(cid * N + sid * OSTRIPE, OSTRIPE)])

        @pl.when(sid == NS - 1)
        def _copy_tail():
            pltpu.sync_copy(acc_sh.at[pl.ds(last, N - last)],
                            f_out.at[pl.ds(cid * N + last, N - last)])
            if with_counts:
                pltpu.sync_copy(acc_cnt.at[pl.ds(last, N - last)],
                                c_out.at[pl.ds(cid * N + last, N - last)])

    return agg


_sc_agg_cnt = _make_sc_agg(True)
_sc_agg = _make_sc_agg(False)

_BN = 1000


def _tc_a(x, wlt, wrt):
    def body(x_ref, wl_ref, wr_ref, p1_ref, r1_ref):
        xb = x_ref[...]
        p1_ref[...] = jnp.dot(xb, wl_ref[...], preferred_element_type=jnp.float32)
        r1_ref[...] = jnp.dot(xb, wr_ref[...], preferred_element_type=jnp.float32)

    return pl.pallas_call(
        body,
        grid=(N // _BN,),
        in_specs=[pl.BlockSpec((_BN, D_IN), lambda i: (i, 0)),
                  pl.BlockSpec((D_IN, D_HID), lambda i: (0, 0)),
                  pl.BlockSpec((D_IN, D_HID), lambda i: (0, 0))],
        out_specs=[pl.BlockSpec((_BN, D_HID), lambda i: (i, 0)),
                   pl.BlockSpec((_BN, D_HID), lambda i: (i, 0))],
        out_shape=[jax.ShapeDtypeStruct((N, D_HID), jnp.float32),
                   jax.ShapeDtypeStruct((N, D_HID), jnp.float32)],
    )(x, wlt, wrt)


def _tc_c(parts1, cnts, r1, b1, w2lt, w2rt, b2):
    def body(p0_ref, p1_ref, c0_ref, c1_ref, r1_ref, b1_ref, wl_ref, wr_ref,
             b2_ref, p2_ref, r2_ref, inv_ref):
        s = p0_ref[...] + p1_ref[...]
        cnt = c0_ref[:, 0:1] + c1_ref[:, 0:1]
        inv = 1.0 / jnp.maximum(cnt, 1.0)
        h1 = s * inv + b1_ref[...] + r1_ref[...]
        p2_ref[...] = jnp.dot(h1, wl_ref[...], preferred_element_type=jnp.float32)
        r2_ref[...] = jnp.dot(h1, wr_ref[...],
                              preferred_element_type=jnp.float32) + b2_ref[...]
        inv_ref[...] = inv

    nb = N // _BN
    return pl.pallas_call(
        body,
        grid=(nb,),
        in_specs=[pl.BlockSpec((_BN, D_HID), lambda i: (i, 0)),
                  pl.BlockSpec((_BN, D_HID), lambda i: (i + nb, 0)),
                  pl.BlockSpec((_BN, 16), lambda i: (i, 0)),
                  pl.BlockSpec((_BN, 16), lambda i: (i + nb, 0)),
                  pl.BlockSpec((_BN, D_HID), lambda i: (i, 0)),
                  pl.BlockSpec((1, D_HID), lambda i: (0, 0)),
                  pl.BlockSpec((D_HID, D_HID), lambda i: (0, 0)),
                  pl.BlockSpec((D_HID, D_HID), lambda i: (0, 0)),
                  pl.BlockSpec((1, D_HID), lambda i: (0, 0))],
        out_specs=[pl.BlockSpec((_BN, D_HID), lambda i: (i, 0)),
                   pl.BlockSpec((_BN, D_HID), lambda i: (i, 0)),
                   pl.BlockSpec((_BN, 1), lambda i: (i, 0))],
        out_shape=[jax.ShapeDtypeStruct((N, D_HID), jnp.float32),
                   jax.ShapeDtypeStruct((N, D_HID), jnp.float32),
                   jax.ShapeDtypeStruct((N, 1), jnp.float32)],
    )(parts1, parts1, cnts, cnts, r1, b1, w2lt, w2rt, b2)


def _tc_e(parts2, r2b, inv):
    nb = N // _BN

    def body(p0_ref, p1_ref, r2_ref, inv_ref, out_ref):
        h2 = (p0_ref[...] + p1_ref[...]) * inv_ref[...] + r2_ref[...]
        h2 = jnp.maximum(h2, 0.0)
        m = jnp.max(h2, axis=1, keepdims=True)
        lse = jnp.log(jnp.sum(jnp.exp(h2 - m), axis=1, keepdims=True)) + m
        out_ref[...] = h2 - lse

    return pl.pallas_call(
        body,
        grid=(nb,),
        in_specs=[pl.BlockSpec((_BN, D_HID), lambda i: (i, 0)),
                  pl.BlockSpec((_BN, D_HID), lambda i: (i + nb, 0)),
                  pl.BlockSpec((_BN, D_HID), lambda i: (i, 0)),
                  pl.BlockSpec((_BN, 1), lambda i: (i, 0))],
        out_specs=pl.BlockSpec((_BN, D_HID), lambda i: (i, 0)),
        out_shape=jax.ShapeDtypeStruct((N, D_HID), jnp.float32),
    )(parts2, parts2, r2b, inv)


def kernel(x, edge_index, W1l, b1l, W1r, W2l, b2l, W2r):
    ei3 = edge_index.reshape(2, NCHUNKS, CHUNK)

    a1, r1 = _tc_a(x, W1l.T, W1r.T)
    parts1, cnts = _sc_agg_cnt(a1, ei3)
    p2, r2b, inv = _tc_c(parts1, cnts, r1, b1l.reshape(1, D_HID),
                         W2l.T, W2r.T, b2l.reshape(1, D_HID))
    parts2, = _sc_agg(p2, ei3)
    return _tc_e(parts2, r2b, inv)


# speedup vs baseline: 1.0618x; 1.0140x over previous
import functools

import jax
import jax.numpy as jnp
from jax import lax
from jax.experimental import pallas as pl
from jax.experimental.pallas import tpu as pltpu
from jax.experimental.pallas import tpu_sc as plsc

N = 10000
E = 160000
D_IN = 1433
D_HID = 32

NC = 2
NS = 16
NW = NC * NS

CHUNK = 128
NCHUNKS = E // CHUNK
CPW = 40
CPW_LAST = NCHUNKS - (NW - 1) * CPW
NBUF = 10
ACC_ROWS = 10112
ZROWS = ACC_ROWS // NS
OSTRIPE = 624


def _make_sc_agg(with_counts):
    mesh = plsc.VectorSubcoreMesh(core_axis_name="c", subcore_axis_name="s")

    out_type = [jax.ShapeDtypeStruct((2 * N, D_HID), jnp.float32)]
    scratch = [
        pltpu.VMEM((CPW, CHUNK), jnp.int32),
        pltpu.VMEM((CPW, CHUNK), jnp.int32),
        [pltpu.VMEM((CHUNK, D_HID), jnp.float32) for _ in range(NBUF)],
        pltpu.VMEM_SHARED((ACC_ROWS, D_HID), jnp.float32),
        [pltpu.SemaphoreType.DMA for _ in range(NBUF)],
        [pltpu.SemaphoreType.DMA for _ in range(NBUF)],
    ]
    if with_counts:
        out_type.append(jax.ShapeDtypeStruct((2 * N, 16), jnp.float32))
        scratch += [
            pltpu.VMEM((CHUNK, 16), jnp.float32),
            pltpu.VMEM((CHUNK, 16), jnp.float32),
            pltpu.VMEM_SHARED((ACC_ROWS, 16), jnp.float32),
            [pltpu.SemaphoreType.DMA for _ in range(NBUF)],
        ]

    @functools.partial(
        pl.kernel,
        out_type=out_type,
        mesh=mesh,
        scratch_types=scratch,
        compiler_params=pltpu.CompilerParams(use_tc_tiling_on_sc=False),
    )
    def agg(table_hbm, ei_hbm, *rest):
        if with_counts:
            (f_out, c_out, src_v, dst_v, rows, acc_sh, gsem, ssem,
             ones_v, z16_v, acc_cnt, csem) = rest
        else:
            f_out, src_v, dst_v, rows, acc_sh, gsem, ssem = rest
        cid = lax.axis_index("c")
        sid = lax.axis_index("s")
        wid = sid * NC + cid
        base = sid * ZROWS

        zv = jnp.zeros((16,), jnp.float32)

        def zrow(r, c):
            rows[0][r, pl.ds(0, 16)] = zv
            rows[0][r, pl.ds(16, 16)] = zv
            if with_counts:
                ones_v[r, pl.ds(0, 16)] = jnp.ones((16,), jnp.float32)
                z16_v[r, pl.ds(0, 16)] = zv
            return c

        lax.fori_loop(0, CHUNK, zrow, 0)
        for k in range(4):
            pltpu.sync_copy(rows[0], acc_sh.at[pl.ds(base + k * CHUNK, CHUNK)])
        pltpu.sync_copy(rows[0].at[pl.ds(0, ZROWS - 4 * CHUNK)],
                        acc_sh.at[pl.ds(base + 4 * CHUNK, ZROWS - 4 * CHUNK)])
        if with_counts:
            for k in range(4):
                pltpu.sync_copy(z16_v, acc_cnt.at[pl.ds(base + k * CHUNK, CHUNK)])
            pltpu.sync_copy(z16_v.at[pl.ds(0, ZROWS - 4 * CHUNK)],
                            acc_cnt.at[pl.ds(base + 4 * CHUNK, ZROWS - 4 * CHUNK)])

        @pl.when(wid < NW - 1)
        def _stage_full():
            pltpu.sync_copy(ei_hbm.at[0, pl.ds(wid * CPW, CPW)], src_v)
            pltpu.sync_copy(ei_hbm.at[1, pl.ds(wid * CPW, CPW)], dst_v)

        @pl.when(wid == NW - 1)
        def _stage_tail():
            pltpu.sync_copy(ei_hbm.at[0, pl.ds((NW - 1) * CPW, CPW_LAST)],
                            src_v.at[pl.ds(0, CPW_LAST)])
            pltpu.sync_copy(ei_hbm.at[1, pl.ds((NW - 1) * CPW, CPW_LAST)],
                            dst_v.at[pl.ds(0, CPW_LAST)])

        plsc.subcore_barrier()

        for b in range(NBUF):
            pltpu.async_copy(table_hbm.at[src_v.at[b]], rows[b], gsem[b])

        n_groups = jnp.where(wid == NW - 1, 1, CPW // NBUF)

        def body(g, carry):
            j0 = g * NBUF
            for b in range(NBUF):
                pltpu.make_async_copy(
                    table_hbm.at[src_v.at[0]], rows[b], gsem[b]).wait()
                pltpu.async_copy(
                    rows[b], acc_sh.at[dst_v.at[j0 + b]], ssem[b], add=True)
                if with_counts:
                    pltpu.async_copy(
                        ones_v, acc_cnt.at[dst_v.at[j0 + b]], csem[b], add=True)

            @pl.when(g < n_groups - 1)
            def _refill():
                for b in range(NBUF):
                    pltpu.make_async_copy(
                        rows[b], acc_sh.at[dst_v.at[0]], ssem[b]).wait()
                    if with_counts:
                        pltpu.make_async_copy(
                            ones_v, acc_cnt.at[dst_v.at[0]], csem[b]).wait()
                    pltpu.async_copy(
                        table_hbm.at[src_v.at[j0 + NBUF + b]], rows[b], gsem[b])
            return carry

        lax.fori_loop(0, n_groups, body, 0)
        for b in range(NBUF):
            pltpu.make_async_copy(rows[b], acc_sh.at[dst_v.at[0]], ssem[b]).wait()
            if with_counts:
                pltpu.make_async_copy(
                    ones_v, acc_cnt.at[dst_v.at[0]], csem[b]).wait()
        plsc.subcore_barrier()

        last = (NS - 1) * OSTRIPE

        @pl.when(sid < NS - 1)
        def _copy_main():
            pltpu.sync_copy(acc_sh.at[pl.ds(sid * OSTRIPE, OSTRIPE)],
                            f_out.at[pl.ds(cid * N + sid * OSTRIPE, OSTRIPE)])
            if with_counts:
                pltpu.sync_copy(acc_cnt.at[pl.ds(sid * OSTRIPE, OSTRIPE)],
                                c_out.at[pl.ds(cid * N + sid * OSTRIPE, OSTRIPE)])

        @pl.when(sid == NS - 1)
        def _copy_tail():
            pltpu.sync_copy(acc_sh.at[pl.ds(last, N - last)],
                            f_out.at[pl.ds(cid * N + last, N - last)])
            if with_counts:
                pltpu.sync_copy(acc_cnt.at[pl.ds(last, N - last)],
                                c_out.at[pl.ds(cid * N + last, N - last)])

    return agg


_sc_agg_cnt = _make_sc_agg(True)
_sc_agg = _make_sc_agg(False)

_BN = 1000


_NT = (((1,), (1,)), ((), ()))


def _tc_a(x, wl, wr):
    def body(x_ref, wl_ref, wr_ref, p1_ref, r1_ref):
        xb = x_ref[...]
        p1_ref[...] = lax.dot_general(xb, wl_ref[...], _NT,
                                      preferred_element_type=jnp.float32)
        r1_ref[...] = lax.dot_general(xb, wr_ref[...], _NT,
                                      preferred_element_type=jnp.float32)

    return pl.pallas_call(
        body,
        grid=(N // _BN,),
        in_specs=[pl.BlockSpec((_BN, D_IN), lambda i: (i, 0)),
                  pl.BlockSpec((D_HID, D_IN), lambda i: (0, 0)),
                  pl.BlockSpec((D_HID, D_IN), lambda i: (0, 0))],
        out_specs=[pl.BlockSpec((_BN, D_HID), lambda i: (i, 0)),
                   pl.BlockSpec((_BN, D_HID), lambda i: (i, 0))],
        out_shape=[jax.ShapeDtypeStruct((N, D_HID), jnp.float32),
                   jax.ShapeDtypeStruct((N, D_HID), jnp.float32)],
    )(x, wl, wr)


def _tc_c(parts1, cnts, r1, b1, w2lt, w2rt, b2):
    def body(p0_ref, p1_ref, c0_ref, c1_ref, r1_ref, b1_ref, wl_ref, wr_ref,
             b2_ref, p2_ref, r2_ref, inv_ref):
        s = p0_ref[...] + p1_ref[...]
        cnt = c0_ref[:, 0:1] + c1_ref[:, 0:1]
        inv = 1.0 / jnp.maximum(cnt, 1.0)
        h1 = s * inv + b1_ref[...] + r1_ref[...]
        p2_ref[...] = lax.dot_general(h1, wl_ref[...], _NT,
                                      preferred_element_type=jnp.float32)
        r2_ref[...] = lax.dot_general(h1, wr_ref[...], _NT,
                                      preferred_element_type=jnp.float32) + b2_ref[...]
        inv_ref[...] = inv

    nb = N // _BN
    return pl.pallas_call(
        body,
        grid=(nb,),
        in_specs=[pl.BlockSpec((_BN, D_HID), lambda i: (i, 0)),
                  pl.BlockSpec((_BN, D_HID), lambda i: (i + nb, 0)),
                  pl.BlockSpec((_BN, 16), lambda i: (i, 0)),
                  pl.BlockSpec((_BN, 16), lambda i: (i + nb, 0)),
                  pl.BlockSpec((_BN, D_HID), lambda i: (i, 0)),
                  pl.BlockSpec((1, D_HID), lambda i: (0, 0)),
                  pl.BlockSpec((D_HID, D_HID), lambda i: (0, 0)),
                  pl.BlockSpec((D_HID, D_HID), lambda i: (0, 0)),
                  pl.BlockSpec((1, D_HID), lambda i: (0, 0))],
        out_specs=[pl.BlockSpec((_BN, D_HID), lambda i: (i, 0)),
                   pl.BlockSpec((_BN, D_HID), lambda i: (i, 0)),
                   pl.BlockSpec((_BN, 1), lambda i: (i, 0))],
        out_shape=[jax.ShapeDtypeStruct((N, D_HID), jnp.float32),
                   jax.ShapeDtypeStruct((N, D_HID), jnp.float32),
                   jax.ShapeDtypeStruct((N, 1), jnp.float32)],
    )(parts1, parts1, cnts, cnts, r1, b1, w2lt, w2rt, b2)


def _tc_e(parts2, r2b, inv):
    nb = N // _BN

    def body(p0_ref, p1_ref, r2_ref, inv_ref, out_ref):
        h2 = (p0_ref[...] + p1_ref[...]) * inv_ref[...] + r2_ref[...]
        h2 = jnp.maximum(h2, 0.0)
        m = jnp.max(h2, axis=1, keepdims=True)
        lse = jnp.log(jnp.sum(jnp.exp(h2 - m), axis=1, keepdims=True)) + m
        out_ref[...] = h2 - lse

    return pl.pallas_call(
        body,
        grid=(nb,),
        in_specs=[pl.BlockSpec((_BN, D_HID), lambda i: (i, 0)),
                  pl.BlockSpec((_BN, D_HID), lambda i: (i + nb, 0)),
                  pl.BlockSpec((_BN, D_HID), lambda i: (i, 0)),
                  pl.BlockSpec((_BN, 1), lambda i: (i, 0))],
        out_specs=pl.BlockSpec((_BN, D_HID), lambda i: (i, 0)),
        out_shape=jax.ShapeDtypeStruct((N, D_HID), jnp.float32),
    )(parts2, parts2, r2b, inv)


def kernel(x, edge_index, W1l, b1l, W1r, W2l, b2l, W2r):
    ei3 = edge_index.reshape(2, NCHUNKS, CHUNK)

    a1, r1 = _tc_a(x, W1l, W1r)
    parts1, cnts = _sc_agg_cnt(a1, ei3)
    p2, r2b, inv = _tc_c(parts1, cnts, r1, b1l.reshape(1, D_HID),
                         W2l, W2r, b2l.reshape(1, D_HID))
    parts2, = _sc_agg(p2, ei3)
    return _tc_e(parts2, r2b, inv)


# speedup vs baseline: 1.0855x; 1.0223x over previous
import functools

import jax
import jax.numpy as jnp
from jax import lax
from jax.experimental import pallas as pl
from jax.experimental.pallas import tpu as pltpu
from jax.experimental.pallas import tpu_sc as plsc

N = 10000
E = 160000
D_IN = 1433
D_HID = 32

NC = 2
NS = 16
NW = NC * NS

CHUNK = 128
NCHUNKS = E // CHUNK
CPW = 40
CPW_LAST = NCHUNKS - (NW - 1) * CPW
NBUF = 10
ACC_ROWS = 10112
ZROWS = ACC_ROWS // NS
OSTRIPE = 624


def _make_sc_agg(with_counts):
    mesh = plsc.VectorSubcoreMesh(core_axis_name="c", subcore_axis_name="s")

    out_type = [jax.ShapeDtypeStruct((2 * N, D_HID), jnp.float32)]
    scratch = [
        pltpu.VMEM((CPW, CHUNK), jnp.int32),
        pltpu.VMEM((CPW, CHUNK), jnp.int32),
        [pltpu.VMEM((CHUNK, D_HID), jnp.float32) for _ in range(NBUF)],
        pltpu.VMEM_SHARED((ACC_ROWS, D_HID), jnp.float32),
        [pltpu.SemaphoreType.DMA for _ in range(NBUF)],
        [pltpu.SemaphoreType.DMA for _ in range(NBUF)],
    ]
    if with_counts:
        out_type.append(jax.ShapeDtypeStruct((2 * N, 16), jnp.float32))
        scratch += [
            pltpu.VMEM((CHUNK, 16), jnp.float32),
            pltpu.VMEM((CHUNK, 16), jnp.float32),
            pltpu.VMEM_SHARED((ACC_ROWS, 16), jnp.float32),
            [pltpu.SemaphoreType.DMA for _ in range(NBUF)],
        ]

    @functools.partial(
        pl.kernel,
        out_type=out_type,
        mesh=mesh,
        scratch_types=scratch,
        compiler_params=pltpu.CompilerParams(use_tc_tiling_on_sc=False),
    )
    def agg(table_hbm, ei_hbm, *rest):
        if with_counts:
            (f_out, c_out, src_v, dst_v, rows, acc_sh, gsem, ssem,
             ones_v, z16_v, acc_cnt, csem) = rest
        else:
            f_out, src_v, dst_v, rows, acc_sh, gsem, ssem = rest
        cid = lax.axis_index("c")
        sid = lax.axis_index("s")
        wid = sid * NC + cid
        base = sid * ZROWS

        zv = jnp.zeros((16,), jnp.float32)

        def zrow(r, c):
            rows[0][r, pl.ds(0, 16)] = zv
            rows[0][r, pl.ds(16, 16)] = zv
            if with_counts:
                ones_v[r, pl.ds(0, 16)] = jnp.ones((16,), jnp.float32)
                z16_v[r, pl.ds(0, 16)] = zv
            return c

        lax.fori_loop(0, CHUNK, zrow, 0)
        for k in range(4):
            pltpu.sync_copy(rows[0], acc_sh.at[pl.ds(base + k * CHUNK, CHUNK)])
        pltpu.sync_copy(rows[0].at[pl.ds(0, ZROWS - 4 * CHUNK)],
                        acc_sh.at[pl.ds(base + 4 * CHUNK, ZROWS - 4 * CHUNK)])
        if with_counts:
            for k in range(4):
                pltpu.sync_copy(z16_v, acc_cnt.at[pl.ds(base + k * CHUNK, CHUNK)])
            pltpu.sync_copy(z16_v.at[pl.ds(0, ZROWS - 4 * CHUNK)],
                            acc_cnt.at[pl.ds(base + 4 * CHUNK, ZROWS - 4 * CHUNK)])

        @pl.when(wid < NW - 1)
        def _stage_full():
            pltpu.sync_copy(ei_hbm.at[0, pl.ds(wid * CPW, CPW)], src_v)
            pltpu.sync_copy(ei_hbm.at[1, pl.ds(wid * CPW, CPW)], dst_v)

        @pl.when(wid == NW - 1)
        def _stage_tail():
            pltpu.sync_copy(ei_hbm.at[0, pl.ds((NW - 1) * CPW, CPW_LAST)],
                            src_v.at[pl.ds(0, CPW_LAST)])
            pltpu.sync_copy(ei_hbm.at[1, pl.ds((NW - 1) * CPW, CPW_LAST)],
                            dst_v.at[pl.ds(0, CPW_LAST)])

        plsc.subcore_barrier()

        for b in range(NBUF):
            pltpu.async_copy(table_hbm.at[src_v.at[b]], rows[b], gsem[b])

        n_groups = jnp.where(wid == NW - 1, 1, CPW // NBUF)

        def body(g, carry):
            j0 = g * NBUF
            for b in range(NBUF):
                pltpu.make_async_copy(
                    table_hbm.at[src_v.at[0]], rows[b], gsem[b]).wait()
                pltpu.async_copy(
                    rows[b], acc_sh.at[dst_v.at[j0 + b]], ssem[b], add=True)
                if with_counts:
                    pltpu.async_copy(
                        ones_v, acc_cnt.at[dst_v.at[j0 + b]], csem[b], add=True)

            @pl.when(g < n_groups - 1)
            def _refill():
                for b in range(NBUF):
                    pltpu.make_async_copy(
                        rows[b], acc_sh.at[dst_v.at[0]], ssem[b]).wait()
                    if with_counts:
                        pltpu.make_async_copy(
                            ones_v, acc_cnt.at[dst_v.at[0]], csem[b]).wait()
                    pltpu.async_copy(
                        table_hbm.at[src_v.at[j0 + NBUF + b]], rows[b], gsem[b])
            return carry

        lax.fori_loop(0, n_groups, body, 0)
        for b in range(NBUF):
            pltpu.make_async_copy(rows[b], acc_sh.at[dst_v.at[0]], ssem[b]).wait()
            if with_counts:
                pltpu.make_async_copy(
                    ones_v, acc_cnt.at[dst_v.at[0]], csem[b]).wait()
        plsc.subcore_barrier()

        last = (NS - 1) * OSTRIPE

        @pl.when(sid < NS - 1)
        def _copy_main():
            pltpu.sync_copy(acc_sh.at[pl.ds(sid * OSTRIPE, OSTRIPE)],
                            f_out.at[pl.ds(cid * N + sid * OSTRIPE, OSTRIPE)])
            if with_counts:
                pltpu.sync_copy(acc_cnt.at[pl.ds(sid * OSTRIPE, OSTRIPE)],
                                c_out.at[pl.ds(cid * N + sid * OSTRIPE, OSTRIPE)])

        @pl.when(sid == NS - 1)
        def _copy_tail():
            pltpu.sync_copy(acc_sh.at[pl.ds(last, N - last)],
                            f_out.at[pl.ds(cid * N + last, N - last)])
            if with_counts:
                pltpu.sync_copy(acc_cnt.at[pl.ds(last, N - last)],
                                c_out.at[pl.ds(cid * N + last, N - last)])

    return agg


_sc_agg_cnt = _make_sc_agg(True)
_sc_agg = _make_sc_agg(False)

_BN = 2000


_NT = (((1,), (1,)), ((), ()))


def _tc_a(x, wl, wr):
    def body(x_ref, wl_ref, wr_ref, p1_ref, r1_ref):
        xb = x_ref[...]
        p1_ref[...] = lax.dot_general(xb, wl_ref[...], _NT,
                                      preferred_element_type=jnp.float32)
        r1_ref[...] = lax.dot_general(xb, wr_ref[...], _NT,
                                      preferred_element_type=jnp.float32)

    return pl.pallas_call(
        body,
        grid=(N // _BN,),
        in_specs=[pl.BlockSpec((_BN, D_IN), lambda i: (i, 0)),
                  pl.BlockSpec((D_HID, D_IN), lambda i: (0, 0)),
                  pl.BlockSpec((D_HID, D_IN), lambda i: (0, 0))],
        out_specs=[pl.BlockSpec((_BN, D_HID), lambda i: (i, 0)),
                   pl.BlockSpec((_BN, D_HID), lambda i: (i, 0))],
        out_shape=[jax.ShapeDtypeStruct((N, D_HID), jnp.float32),
                   jax.ShapeDtypeStruct((N, D_HID), jnp.float32)],
    )(x, wl, wr)


def _tc_c(parts1, cnts, r1, b1, w2lt, w2rt, b2):
    def body(p0_ref, p1_ref, c0_ref, c1_ref, r1_ref, b1_ref, wl_ref, wr_ref,
             b2_ref, p2_ref, r2_ref, inv_ref):
        s = p0_ref[...] + p1_ref[...]
        cnt = c0_ref[:, 0:1] + c1_ref[:, 0:1]
        inv = 1.0 / jnp.maximum(cnt, 1.0)
        h1 = s * inv + b1_ref[...] + r1_ref[...]
        p2_ref[...] = lax.dot_general(h1, wl_ref[...], _NT,
                                      preferred_element_type=jnp.float32)
        r2_ref[...] = lax.dot_general(h1, wr_ref[...], _NT,
                                      preferred_element_type=jnp.float32) + b2_ref[...]
        inv_ref[...] = inv

    nb = N // _BN
    return pl.pallas_call(
        body,
        grid=(nb,),
        in_specs=[pl.BlockSpec((_BN, D_HID), lambda i: (i, 0)),
                  pl.BlockSpec((_BN, D_HID), lambda i: (i + nb, 0)),
                  pl.BlockSpec((_BN, 16), lambda i: (i, 0)),
                  pl.BlockSpec((_BN, 16), lambda i: (i + nb, 0)),
                  pl.BlockSpec((_BN, D_HID), lambda i: (i, 0)),
                  pl.BlockSpec((1, D_HID), lambda i: (0, 0)),
                  pl.BlockSpec((D_HID, D_HID), lambda i: (0, 0)),
                  pl.BlockSpec((D_HID, D_HID), lambda i: (0, 0)),
                  pl.BlockSpec((1, D_HID), lambda i: (0, 0))],
        out_specs=[pl.BlockSpec((_BN, D_HID), lambda i: (i, 0)),
                   pl.BlockSpec((_BN, D_HID), lambda i: (i, 0)),
                   pl.BlockSpec((_BN, 1), lambda i: (i, 0))],
        out_shape=[jax.ShapeDtypeStruct((N, D_HID), jnp.float32),
                   jax.ShapeDtypeStruct((N, D_HID), jnp.float32),
                   jax.ShapeDtypeStruct((N, 1), jnp.float32)],
    )(parts1, parts1, cnts, cnts, r1, b1, w2lt, w2rt, b2)


def _tc_e(parts2, r2b, inv):
    nb = N // _BN

    def body(p0_ref, p1_ref, r2_ref, inv_ref, out_ref):
        h2 = (p0_ref[...] + p1_ref[...]) * inv_ref[...] + r2_ref[...]
        h2 = jnp.maximum(h2, 0.0)
        m = jnp.max(h2, axis=1, keepdims=True)
        lse = jnp.log(jnp.sum(jnp.exp(h2 - m), axis=1, keepdims=True)) + m
        out_ref[...] = h2 - lse

    return pl.pallas_call(
        body,
        grid=(nb,),
        in_specs=[pl.BlockSpec((_BN, D_HID), lambda i: (i, 0)),
                  pl.BlockSpec((_BN, D_HID), lambda i: (i + nb, 0)),
                  pl.BlockSpec((_BN, D_HID), lambda i: (i, 0)),
                  pl.BlockSpec((_BN, 1), lambda i: (i, 0))],
        out_specs=pl.BlockSpec((_BN, D_HID), lambda i: (i, 0)),
        out_shape=jax.ShapeDtypeStruct((N, D_HID), jnp.float32),
    )(parts2, parts2, r2b, inv)


def kernel(x, edge_index, W1l, b1l, W1r, W2l, b2l, W2r):
    ei3 = edge_index.reshape(2, NCHUNKS, CHUNK)

    a1, r1 = _tc_a(x, W1l, W1r)
    parts1, cnts = _sc_agg_cnt(a1, ei3)
    p2, r2b, inv = _tc_c(parts1, cnts, r1, b1l.reshape(1, D_HID),
                         W2l, W2r, b2l.reshape(1, D_HID))
    parts2, = _sc_agg(p2, ei3)
    return _tc_e(parts2, r2b, inv)


# speedup vs baseline: 1.0888x; 1.0031x over previous
import functools

import jax
import jax.numpy as jnp
from jax import lax
from jax.experimental import pallas as pl
from jax.experimental.pallas import tpu as pltpu
from jax.experimental.pallas import tpu_sc as plsc

N = 10000
E = 160000
D_IN = 1433
D_HID = 32

NC = 2
NS = 16
NW = NC * NS

CHUNK = 128
NCHUNKS = E // CHUNK
CPW = 40
CPW_LAST = NCHUNKS - (NW - 1) * CPW
NBUF = 10
ACC_ROWS = 10112
ZROWS = ACC_ROWS // NS
OSTRIPE = 624


def _make_sc_agg(with_counts):
    mesh = plsc.VectorSubcoreMesh(core_axis_name="c", subcore_axis_name="s")

    out_type = [jax.ShapeDtypeStruct((2 * N, D_HID), jnp.float32)]
    scratch = [
        pltpu.VMEM((CPW, CHUNK), jnp.int32),
        pltpu.VMEM((CPW, CHUNK), jnp.int32),
        [pltpu.VMEM((CHUNK, D_HID), jnp.float32) for _ in range(NBUF)],
        pltpu.VMEM_SHARED((ACC_ROWS, D_HID), jnp.float32),
        [pltpu.SemaphoreType.DMA for _ in range(NBUF)],
        [pltpu.SemaphoreType.DMA for _ in range(NBUF)],
    ]
    if with_counts:
        out_type.append(jax.ShapeDtypeStruct((2 * N, 16), jnp.float32))
        scratch += [
            pltpu.VMEM((CHUNK, 16), jnp.float32),
            pltpu.VMEM((CHUNK, 16), jnp.float32),
            pltpu.VMEM_SHARED((ACC_ROWS, 16), jnp.float32),
            [pltpu.SemaphoreType.DMA for _ in range(NBUF)],
        ]

    @functools.partial(
        pl.kernel,
        out_type=out_type,
        mesh=mesh,
        scratch_types=scratch,
        compiler_params=pltpu.CompilerParams(use_tc_tiling_on_sc=False),
    )
    def agg(table_hbm, ei_hbm, *rest):
        if with_counts:
            (f_out, c_out, src_v, dst_v, rows, acc_sh, gsem, ssem,
             ones_v, z16_v, acc_cnt, csem) = rest
        else:
            f_out, src_v, dst_v, rows, acc_sh, gsem, ssem = rest
        cid = lax.axis_index("c")
        sid = lax.axis_index("s")
        wid = sid * NC + cid
        base = sid * ZROWS

        zv = jnp.zeros((16,), jnp.float32)

        def zrow(r, c):
            rows[0][r, pl.ds(0, 16)] = zv
            rows[0][r, pl.ds(16, 16)] = zv
            if with_counts:
                ones_v[r, pl.ds(0, 16)] = jnp.ones((16,), jnp.float32)
                z16_v[r, pl.ds(0, 16)] = zv
            return c

        lax.fori_loop(0, CHUNK, zrow, 0)
        for k in range(4):
            pltpu.sync_copy(rows[0], acc_sh.at[pl.ds(base + k * CHUNK, CHUNK)])
        pltpu.sync_copy(rows[0].at[pl.ds(0, ZROWS - 4 * CHUNK)],
                        acc_sh.at[pl.ds(base + 4 * CHUNK, ZROWS - 4 * CHUNK)])
        if with_counts:
            for k in range(4):
                pltpu.sync_copy(z16_v, acc_cnt.at[pl.ds(base + k * CHUNK, CHUNK)])
            pltpu.sync_copy(z16_v.at[pl.ds(0, ZROWS - 4 * CHUNK)],
                            acc_cnt.at[pl.ds(base + 4 * CHUNK, ZROWS - 4 * CHUNK)])

        @pl.when(wid < NW - 1)
        def _stage_full():
            pltpu.sync_copy(ei_hbm.at[0, pl.ds(wid * CPW, CPW)], src_v)
            pltpu.sync_copy(ei_hbm.at[1, pl.ds(wid * CPW, CPW)], dst_v)

        @pl.when(wid == NW - 1)
        def _stage_tail():
            pltpu.sync_copy(ei_hbm.at[0, pl.ds((NW - 1) * CPW, CPW_LAST)],
                            src_v.at[pl.ds(0, CPW_LAST)])
            pltpu.sync_copy(ei_hbm.at[1, pl.ds((NW - 1) * CPW, CPW_LAST)],
                            dst_v.at[pl.ds(0, CPW_LAST)])

        plsc.subcore_barrier()

        for b in range(NBUF):
            pltpu.async_copy(table_hbm.at[src_v.at[b]], rows[b], gsem[b])

        n_groups = jnp.where(wid == NW - 1, 1, CPW // NBUF)

        def body(g, carry):
            j0 = g * NBUF
            for b in range(NBUF):
                pltpu.make_async_copy(
                    table_hbm.at[src_v.at[0]], rows[b], gsem[b]).wait()
                pltpu.async_copy(
                    rows[b], acc_sh.at[dst_v.at[j0 + b]], ssem[b], add=True)
                if with_counts:
                    pltpu.async_copy(
                        ones_v, acc_cnt.at[dst_v.at[j0 + b]], csem[b], add=True)

            @pl.when(g < n_groups - 1)
            def _refill():
                for b in range(NBUF):
                    pltpu.make_async_copy(
                        rows[b], acc_sh.at[dst_v.at[0]], ssem[b]).wait()
                    if with_counts:
                        pltpu.make_async_copy(
                            ones_v, acc_cnt.at[dst_v.at[0]], csem[b]).wait()
                    pltpu.async_copy(
                        table_hbm.at[src_v.at[j0 + NBUF + b]], rows[b], gsem[b])
            return carry

        lax.fori_loop(0, n_groups, body, 0)
        for b in range(NBUF):
            pltpu.make_async_copy(rows[b], acc_sh.at[dst_v.at[0]], ssem[b]).wait()
            if with_counts:
                pltpu.make_async_copy(
                    ones_v, acc_cnt.at[dst_v.at[0]], csem[b]).wait()
        plsc.subcore_barrier()

        last = (NS - 1) * OSTRIPE

        @pl.when(sid < NS - 1)
        def _copy_main():
            pltpu.sync_copy(acc_sh.at[pl.ds(sid * OSTRIPE, OSTRIPE)],
                            f_out.at[pl.ds(cid * N + sid * OSTRIPE, OSTRIPE)])
            if with_counts:
                pltpu.sync_copy(acc_cnt.at[pl.ds(sid * OSTRIPE, OSTRIPE)],
                                c_out.at[pl.ds(cid * N + sid * OSTRIPE, OSTRIPE)])

        @pl.when(sid == NS - 1)
        def _copy_tail():
            pltpu.sync_copy(acc_sh.at[pl.ds(last, N - last)],
                            f_out.at[pl.ds(cid * N + last, N - last)])
            if with_counts:
                pltpu.sync_copy(acc_cnt.at[pl.ds(last, N - last)],
                                c_out.at[pl.ds(cid * N + last, N - last)])

    return agg


_sc_agg_cnt = _make_sc_agg(True)
_sc_agg = _make_sc_agg(False)

_BN = 2000


_NT = (((1,), (1,)), ((), ()))


def _tc_a(x, wl, wr):
    def body(x_ref, wl_ref, wr_ref, p1_ref, r1_ref):
        xb = x_ref[...]
        p1_ref[...] = lax.dot_general(xb, wl_ref[...], _NT,
                                      preferred_element_type=jnp.float32)
        r1_ref[...] = lax.dot_general(xb, wr_ref[...], _NT,
                                      preferred_element_type=jnp.float32)

    return pl.pallas_call(
        body,
        grid=(N // _BN,),
        in_specs=[pl.BlockSpec((_BN, D_IN), lambda i: (i, 0)),
                  pl.BlockSpec((D_HID, D_IN), lambda i: (0, 0)),
                  pl.BlockSpec((D_HID, D_IN), lambda i: (0, 0))],
        out_specs=[pl.BlockSpec((_BN, D_HID), lambda i: (i, 0)),
                   pl.BlockSpec((_BN, D_HID), lambda i: (i, 0))],
        out_shape=[jax.ShapeDtypeStruct((N, D_HID), jnp.float32),
                   jax.ShapeDtypeStruct((N, D_HID), jnp.float32)],
    )(x, wl, wr)


def _tc_c(parts1, cnts, r1, b1, w2lt, w2rt, b2):
    def body(p0_ref, p1_ref, c0_ref, c1_ref, r1_ref, b1_ref, wl_ref, wr_ref,
             b2_ref, p2_ref, r2_ref):
        s = p0_ref[...] + p1_ref[...]
        cnt = c0_ref[:, 0:1] + c1_ref[:, 0:1]
        inv = 1.0 / jnp.maximum(cnt, 1.0)
        h1 = s * inv + b1_ref[...] + r1_ref[...]
        p2_ref[...] = lax.dot_general(h1, wl_ref[...], _NT,
                                      preferred_element_type=jnp.float32)
        r2_ref[...] = lax.dot_general(h1, wr_ref[...], _NT,
                                      preferred_element_type=jnp.float32) + b2_ref[...]

    nb = N // _BN
    return pl.pallas_call(
        body,
        grid=(nb,),
        in_specs=[pl.BlockSpec((_BN, D_HID), lambda i: (i, 0)),
                  pl.BlockSpec((_BN, D_HID), lambda i: (i + nb, 0)),
                  pl.BlockSpec((_BN, 16), lambda i: (i, 0)),
                  pl.BlockSpec((_BN, 16), lambda i: (i + nb, 0)),
                  pl.BlockSpec((_BN, D_HID), lambda i: (i, 0)),
                  pl.BlockSpec((1, D_HID), lambda i: (0, 0)),
                  pl.BlockSpec((D_HID, D_HID), lambda i: (0, 0)),
                  pl.BlockSpec((D_HID, D_HID), lambda i: (0, 0)),
                  pl.BlockSpec((1, D_HID), lambda i: (0, 0))],
        out_specs=[pl.BlockSpec((_BN, D_HID), lambda i: (i, 0)),
                   pl.BlockSpec((_BN, D_HID), lambda i: (i, 0))],
        out_shape=[jax.ShapeDtypeStruct((N, D_HID), jnp.float32),
                   jax.ShapeDtypeStruct((N, D_HID), jnp.float32)],
    )(parts1, parts1, cnts, cnts, r1, b1, w2lt, w2rt, b2)


def _tc_e(parts2, cnts, r2b):
    nb = N // _BN

    def body(p0_ref, p1_ref, c0_ref, c1_ref, r2_ref, out_ref):
        cnt = c0_ref[:, 0:1] + c1_ref[:, 0:1]
        inv = 1.0 / jnp.maximum(cnt, 1.0)
        h2 = (p0_ref[...] + p1_ref[...]) * inv + r2_ref[...]
        h2 = jnp.maximum(h2, 0.0)
        m = jnp.max(h2, axis=1, keepdims=True)
        lse = jnp.log(jnp.sum(jnp.exp(h2 - m), axis=1, keepdims=True)) + m
        out_ref[...] = h2 - lse

    return pl.pallas_call(
        body,
        grid=(nb,),
        in_specs=[pl.BlockSpec((_BN, D_HID), lambda i: (i, 0)),
                  pl.BlockSpec((_BN, D_HID), lambda i: (i + nb, 0)),
                  pl.BlockSpec((_BN, 16), lambda i: (i, 0)),
                  pl.BlockSpec((_BN, 16), lambda i: (i + nb, 0)),
                  pl.BlockSpec((_BN, D_HID), lambda i: (i, 0))],
        out_specs=pl.BlockSpec((_BN, D_HID), lambda i: (i, 0)),
        out_shape=jax.ShapeDtypeStruct((N, D_HID), jnp.float32),
    )(parts2, parts2, cnts, cnts, r2b)


def kernel(x, edge_index, W1l, b1l, W1r, W2l, b2l, W2r):
    ei3 = edge_index.reshape(2, NCHUNKS, CHUNK)

    a1, r1 = _tc_a(x, W1l, W1r)
    parts1, cnts = _sc_agg_cnt(a1, ei3)
    p2, r2b = _tc_c(parts1, cnts, r1, b1l.reshape(1, D_HID),
                    W2l, W2r, b2l.reshape(1, D_HID))
    parts2, = _sc_agg(p2, ei3)
    return _tc_e(parts2, cnts, r2b)
